# baseline (speedup 1.0000x reference)
import functools

import jax
import jax.numpy as jnp
from jax import lax
from jax.experimental import pallas as pl
from jax.experimental.pallas import tpu as pltpu
from jax.experimental.pallas import tpu_sc as plsc

_NCLASSES = 19
_S = 7
_K = _S * _S
_C = 32
_H, _W = 64, 2048
_PAD = (_S - 1) // 2
_HP, _WP = _H + 2 * _PAD, _W + 2 * _PAD
_N = 16384
_ROWS = _K * _N
_IDX_MINOR = 98
_IDX_MAJOR = _ROWS // _IDX_MINOR
_CHUNK_GROUPS = 8
_CHUNK = _CHUNK_GROUPS * _IDX_MINOR
_NTILES = 32
_ROWS_PER_TILE = _ROWS // _NTILES
_CHUNKS_PER_TILE = _ROWS_PER_TILE // _CHUNK

_PPACK = 4
_Q = _N // _PPACK
_QBLK = 64
_GRID = _Q // _QBLK
_R = _K * _QBLK
_OL = _PPACK * _NCLASSES


def _sc_gather(idx2, tx, tp):
    mesh = plsc.VectorSubcoreMesh(core_axis_name="c", subcore_axis_name="s")

    @functools.partial(
        pl.kernel,
        mesh=mesh,
        compiler_params=pltpu.CompilerParams(use_tc_tiling_on_sc=False),
        out_type=[
            jax.ShapeDtypeStruct((_ROWS, _C), jnp.float32),
            jax.ShapeDtypeStruct((_ROWS, _C), jnp.float32),
        ],
        scratch_types=[
            pltpu.VMEM((_CHUNK_GROUPS, _IDX_MINOR), jnp.int32),
            pltpu.VMEM((_CHUNK, _C), jnp.float32),
            pltpu.VMEM((_CHUNK, _C), jnp.float32),
            pltpu.SemaphoreType.DMA,
        ],
    )
    def k(idx_hbm, tx_hbm, tp_hbm, gx_hbm, gp_hbm, idx_v, gx_v, gp_v, sem):
        wid = lax.axis_index("s") * 2 + lax.axis_index("c")
        idx_row0 = wid * (_ROWS_PER_TILE // _IDX_MINOR)
        row0 = wid * _ROWS_PER_TILE

        def body(c, _):
            pltpu.sync_copy(
                idx_hbm.at[pl.ds(idx_row0 + c * _CHUNK_GROUPS, _CHUNK_GROUPS)],
                idx_v)
            handles = []
            for j in range(_CHUNK_GROUPS):
                handles.append(pltpu.async_copy(
                    tx_hbm.at[idx_v.at[j]],
                    gx_v.at[pl.ds(j * _IDX_MINOR, _IDX_MINOR)], sem))
                handles.append(pltpu.async_copy(
                    tp_hbm.at[idx_v.at[j]],
                    gp_v.at[pl.ds(j * _IDX_MINOR, _IDX_MINOR)], sem))
            for h in handles:
                h.wait()
            base = row0 + c * _CHUNK
            pltpu.sync_copy(gx_v, gx_hbm.at[pl.ds(base, _CHUNK)])
            pltpu.sync_copy(gp_v, gp_hbm.at[pl.ds(base, _CHUNK)])
            return _

        lax.fori_loop(0, _CHUNKS_PER_TILE, body, None)

    return k(idx2, tx, tp)


def _tc_body(gx_ref, gp_ref, up_ref, fc1_ref, b1_ref, lnw_ref, lnb_ref,
             g_ref, bg_ref, fc2_ref, b2_ref, s_ref, bs_ref, conv_ref, cb_ref,
             out_ref):
    gp3 = gp_ref[...]
    up = up_ref[...]
    d = jnp.abs(gp3 - up[None, :, :]).reshape(_R, 128)
    hc = jnp.dot(d, fc1_ref[...], preferred_element_type=jnp.float32, precision=lax.Precision.HIGHEST) + b1_ref[...]
    vs = jnp.dot(hc * hc, g_ref[...], preferred_element_type=jnp.float32, precision=lax.Precision.HIGHEST)
    inv = jnp.dot(lax.rsqrt(vs + 1e-5), bg_ref[...],
                  preferred_element_type=jnp.float32, precision=lax.Precision.HIGHEST)
    h = hc * inv * lnw_ref[...] + lnb_ref[...]
    h = 0.5 * h * (1.0 + lax.erf(h * 0.7071067811865476))
    h = jnp.dot(h, fc2_ref[...], preferred_element_type=jnp.float32, precision=lax.Precision.HIGHEST) + b2_ref[...]
    e = jnp.exp(h)
    s = jnp.dot(e, s_ref[...], preferred_element_type=jnp.float32, precision=lax.Precision.HIGHEST)
    w = e * jnp.dot(1.0 / s, bs_ref[...], preferred_element_type=jnp.float32, precision=lax.Precision.HIGHEST)
    wx = gx_ref[...].reshape(_R, 128) * w
    wx3 = wx.reshape(_K, _QBLK, 128)
    res = lax.dot_general(
        wx3, conv_ref[...],
        dimension_numbers=(((2,), (1,)), ((0,), (0,))),
        preferred_element_type=jnp.float32,
        precision=lax.Precision.HIGHEST)
    out_ref[...] = jnp.sum(res, axis=0) + cb_ref[...]


def _tc_mlp(gx3, gp3, up128, fc1bd, b1c, lnw, lnb, gmat, bg, fc2bd, b2t,
            smat, bs, conv4, cb):
    return pl.pallas_call(
        _tc_body,
        grid=(_GRID,),
        in_specs=[
            pl.BlockSpec((_K, _QBLK, 128), lambda i: (0, i, 0)),
            pl.BlockSpec((_K, _QBLK, 128), lambda i: (0, i, 0)),
            pl.BlockSpec((_QBLK, 128), lambda i: (i, 0)),
            pl.BlockSpec((128, 256), lambda i: (0, 0)),
            pl.BlockSpec((256,), lambda i: (0,)),
            pl.BlockSpec((256,), lambda i: (0,)),
            pl.BlockSpec((256,), lambda i: (0,)),
            pl.BlockSpec((256, _PPACK), lambda i: (0, 0)),
            pl.BlockSpec((_PPACK, 256), lambda i: (0, 0)),
            pl.BlockSpec((256, 128), lambda i: (0, 0)),
            pl.BlockSpec((128,), lambda i: (0,)),
            pl.BlockSpec((128, _PPACK), lambda i: (0, 0)),
            pl.BlockSpec((_PPACK, 128), lambda i: (0, 0)),
            pl.BlockSpec((_K, 128, _OL), lambda i: (0, 0, 0)),
            pl.BlockSpec((1, _OL), lambda i: (0, 0)),
        ],
        out_specs=pl.BlockSpec((_QBLK, _OL), lambda i: (i, 0)),
        out_shape=jax.ShapeDtypeStruct((_Q, _OL), jnp.float32),
    )(gx3, gp3, up128, fc1bd, b1c, lnw, lnb, gmat, bg, fc2bd, b2t,
      smat, bs, conv4, cb)


def kernel(x, proj_range_xyz, unproj_range_xyz, p2ri_lut, num_valid_pts,
           fc1_w, fc1_b, ln_w, ln_b, fc2_w, fc2_b, conv_w, conv_b):
    f32 = jnp.float32
    xp = jnp.pad(x[0], ((0, 0), (_PAD, _PAD), (_PAD, _PAD)))
    pp = jnp.pad(proj_range_xyz[0], ((0, 0), (_PAD, _PAD), (_PAD, _PAD)))
    tx = jnp.transpose(xp, (1, 2, 0)).reshape(_HP * _WP, _C)
    tp = jnp.transpose(pp, (1, 2, 0))
    tp = jnp.pad(tp, ((0, 0), (0, 0), (0, _C - 4))).reshape(_HP * _WP, _C)

    lut = p2ri_lut[0]
    yc = lut[:, 1]
    xc = lut[:, 2]
    dy = jnp.arange(_S, dtype=jnp.int32)
    off = (dy[:, None] * _WP + dy[None, :]).reshape(_K)
    base = yc * _WP + xc
    idx = (off[:, None] + base[None, :]).reshape(_ROWS)
    idx2 = idx.reshape(_IDX_MAJOR, _IDX_MINOR).astype(jnp.int32)

    up128 = jnp.pad(unproj_range_xyz[0], ((0, 0), (0, _C - 4))).reshape(_Q, 128)

    eye4 = jnp.eye(_PPACK, dtype=f32)
    fc1p = jnp.pad(fc1_w.T, ((0, _C - 4), (0, 0)))
    fc1c = fc1p - jnp.mean(fc1p, axis=1, keepdims=True)
    fc1bd = jnp.kron(eye4, fc1c)
    b1c = jnp.tile(fc1_b - jnp.mean(fc1_b), _PPACK)
    lnw = jnp.tile(ln_w, _PPACK)
    lnb = jnp.tile(ln_b, _PPACK)
    gmat = jnp.kron(eye4, jnp.full((64, 1), 1.0 / 64, f32))
    bg4 = jnp.kron(eye4, jnp.ones((1, 64), f32))
    fc2bd = jnp.kron(eye4, fc2_w.T)
    b2t = jnp.tile(fc2_b, _PPACK)
    smat = jnp.kron(eye4, jnp.ones((_C, 1), f32))
    bs4 = jnp.kron(eye4, jnp.ones((1, _C), f32))
    convr3 = jnp.transpose(conv_w.reshape(_NCLASSES, _C, _K), (2, 1, 0))
    conv4 = jax.vmap(lambda m: jnp.kron(eye4, m))(convr3)
    cb = jnp.tile(conv_b, _PPACK)[None]

    gx, gp = _sc_gather(idx2, tx, tp)
    gx3 = gx.reshape(_K, _Q, 128)
    gp3 = gp.reshape(_K, _Q, 128)

    out = _tc_mlp(gx3, gp3, up128, fc1bd, b1c, lnw, lnb, gmat, bg4, fc2bd,
                  b2t, smat, bs4, conv4, cb)
    out = out.reshape(_Q, _PPACK, _NCLASSES).transpose(2, 0, 1).reshape(
        _NCLASSES, _N)
    return out[None]


# speedup vs baseline: 3.6784x; 3.6784x over previous
import functools

import jax
import jax.numpy as jnp
from jax import lax
from jax.experimental import pallas as pl
from jax.experimental.pallas import tpu as pltpu
from jax.experimental.pallas import tpu_sc as plsc

_NCLASSES = 19
_S = 7
_K = _S * _S
_C = 32
_H, _W = 64, 2048
_PAD = (_S - 1) // 2
_HP = _H + 2 * _PAD
_WPP = 2064
_NPIX = _HP * _WPP
_N = 16384
_NH = _N // 2
_ROWSH = _K * _NH
_IDX_MINOR = 98
_CHUNK_GROUPS = 8
_CHUNK = _CHUNK_GROUPS * _IDX_MINOR
_NTILES = 32

_PPACK = 4
_QH = _NH // _PPACK
_QBLK = 64
_R = _K * _QBLK
_OL = _PPACK * _NCLASSES


def _sc_gather(idx2, tx, tp):
    rows = idx2.shape[0] * _IDX_MINOR
    rows_per_tile = rows // _NTILES
    chunks_per_tile = rows_per_tile // _CHUNK
    mesh = plsc.VectorSubcoreMesh(core_axis_name="c", subcore_axis_name="s")

    @functools.partial(
        pl.kernel,
        mesh=mesh,
        compiler_params=pltpu.CompilerParams(use_tc_tiling_on_sc=False),
        out_type=[
            jax.ShapeDtypeStruct((rows, _C), jnp.float32),
            jax.ShapeDtypeStruct((rows, _C), jnp.float32),
        ],
        scratch_types=[
            pltpu.VMEM((_CHUNK_GROUPS, _IDX_MINOR), jnp.int32),
            pltpu.VMEM((_CHUNK, _C), jnp.float32),
            pltpu.VMEM((_CHUNK, _C), jnp.float32),
            pltpu.SemaphoreType.DMA,
        ],
    )
    def k(idx_hbm, tx_hbm, tp_hbm, gx_hbm, gp_hbm, idx_v, gx_v, gp_v, sem):
        wid = lax.axis_index("s") * 2 + lax.axis_index("c")
        idx_row0 = wid * (rows_per_tile // _IDX_MINOR)
        row0 = wid * rows_per_tile

        def body(c, _):
            pltpu.sync_copy(
                idx_hbm.at[pl.ds(idx_row0 + c * _CHUNK_GROUPS, _CHUNK_GROUPS)],
                idx_v)
            handles = []
            for j in range(_CHUNK_GROUPS):
                handles.append(pltpu.async_copy(
                    tx_hbm.at[idx_v.at[j]],
                    gx_v.at[pl.ds(j * _IDX_MINOR, _IDX_MINOR)], sem))
                handles.append(pltpu.async_copy(
                    tp_hbm.at[idx_v.at[j]],
                    gp_v.at[pl.ds(j * _IDX_MINOR, _IDX_MINOR)], sem))
            for h in handles:
                h.wait()
            base = row0 + c * _CHUNK
            pltpu.sync_copy(gx_v, gx_hbm.at[pl.ds(base, _CHUNK)])
            pltpu.sync_copy(gp_v, gp_hbm.at[pl.ds(base, _CHUNK)])
            return _

        lax.fori_loop(0, chunks_per_tile, body, None)

    return k(idx2, tx, tp)


def _tc_body(gx_ref, gp_ref, up_ref, fc1_ref, b1_ref, lnw_ref, lnb_ref,
             g_ref, bg_ref, fc2_ref, b2_ref, s_ref, bs_ref, conv_ref, cb_ref,
             out_ref):
    gp3 = gp_ref[...]
    up = up_ref[...]
    d = jnp.abs(gp3 - up[None, :, :]).reshape(_R, 128)
    hc = jnp.dot(d, fc1_ref[...], preferred_element_type=jnp.float32) + b1_ref[...]
    vs = jnp.dot(hc * hc, g_ref[...], preferred_element_type=jnp.float32)
    inv = jnp.dot(lax.rsqrt(vs + 1e-5), bg_ref[...],
                  preferred_element_type=jnp.float32)
    h = hc * inv * lnw_ref[...] + lnb_ref[...]
    h = 0.5 * h * (1.0 + lax.erf(h * 0.7071067811865476))
    h = jnp.dot(h, fc2_ref[...], preferred_element_type=jnp.float32) + b2_ref[...]
    e = jnp.exp(h)
    s = jnp.dot(e, s_ref[...], preferred_element_type=jnp.float32)
    w = e * jnp.dot(1.0 / s, bs_ref[...], preferred_element_type=jnp.float32)
    wx = gx_ref[...].reshape(_R, 128) * w
    wx3 = wx.reshape(_K, _QBLK, 128)
    res = lax.dot_general(
        wx3, conv_ref[...],
        dimension_numbers=(((2,), (1,)), ((0,), (0,))),
        preferred_element_type=jnp.float32)
    out_ref[...] = jnp.sum(res, axis=0) + cb_ref[...]


def _tc_mlp(gx3, gp3, up128, weights):
    q = up128.shape[0]
    return pl.pallas_call(
        _tc_body,
        grid=(q // _QBLK,),
        in_specs=[
            pl.BlockSpec((_K, _QBLK, 128), lambda i: (0, i, 0)),
            pl.BlockSpec((_K, _QBLK, 128), lambda i: (0, i, 0)),
            pl.BlockSpec((_QBLK, 128), lambda i: (i, 0)),
            pl.BlockSpec((128, 256), lambda i: (0, 0)),
            pl.BlockSpec((256,), lambda i: (0,)),
            pl.BlockSpec((256,), lambda i: (0,)),
            pl.BlockSpec((256,), lambda i: (0,)),
            pl.BlockSpec((256, _PPACK), lambda i: (0, 0)),
            pl.BlockSpec((_PPACK, 256), lambda i: (0, 0)),
            pl.BlockSpec((256, 128), lambda i: (0, 0)),
            pl.BlockSpec((128,), lambda i: (0,)),
            pl.BlockSpec((128, _PPACK), lambda i: (0, 0)),
            pl.BlockSpec((_PPACK, 128), lambda i: (0, 0)),
            pl.BlockSpec((_K, 128, _OL), lambda i: (0, 0, 0)),
            pl.BlockSpec((1, _OL), lambda i: (0, 0)),
        ],
        out_specs=pl.BlockSpec((_QBLK, _OL), lambda i: (i, 0)),
        out_shape=jax.ShapeDtypeStruct((q, _OL), jnp.float32),
    )(gx3, gp3, up128, *weights)


def kernel(x, proj_range_xyz, unproj_range_xyz, p2ri_lut, num_valid_pts,
           fc1_w, fc1_b, ln_w, ln_b, fc2_w, fc2_b, conv_w, conv_b):
    f32 = jnp.float32
    wpad = _WPP - _W - _PAD
    xp = jnp.pad(x[0], ((0, 0), (_PAD, _PAD), (_PAD, wpad)))
    pp = jnp.pad(proj_range_xyz[0], ((0, 0), (_PAD, _PAD), (_PAD, wpad)))
    tx = jnp.transpose(xp, (1, 2, 0)).reshape(_NPIX // 4, 128)
    tx = tx.reshape(_NPIX, _C)
    tp = jnp.transpose(pp, (1, 2, 0))
    tp = jnp.pad(tp, ((0, 0), (0, 0), (0, _C - 4)))
    tp = tp.reshape(_NPIX // 4, 128).reshape(_NPIX, _C)

    lut = p2ri_lut[0]
    yc = lut[:, 1]
    xc = lut[:, 2]
    dy = jnp.arange(_S, dtype=jnp.int32)
    off = (dy[:, None] * _WPP + dy[None, :]).reshape(_K)
    base = yc * _WPP + xc
    idx = off[:, None] + base[None, :]
    idx2a = idx[:, :_NH].reshape(-1, _IDX_MINOR).astype(jnp.int32)
    idx2b = idx[:, _NH:].reshape(-1, _IDX_MINOR).astype(jnp.int32)

    up128 = jnp.pad(unproj_range_xyz[0], ((0, 0), (0, _C - 4))).reshape(-1, 128)

    eye4 = jnp.eye(_PPACK, dtype=f32)
    fc1p = jnp.pad(fc1_w.T, ((0, _C - 4), (0, 0)))
    fc1c = fc1p - jnp.mean(fc1p, axis=1, keepdims=True)
    fc1bd = jnp.kron(eye4, fc1c)
    b1c = jnp.tile(fc1_b - jnp.mean(fc1_b), _PPACK)
    lnw = jnp.tile(ln_w, _PPACK)
    lnb = jnp.tile(ln_b, _PPACK)
    gmat = jnp.kron(eye4, jnp.full((64, 1), 1.0 / 64, f32))
    bg4 = jnp.kron(eye4, jnp.ones((1, 64), f32))
    fc2bd = jnp.kron(eye4, fc2_w.T)
    b2t = jnp.tile(fc2_b, _PPACK)
    smat = jnp.kron(eye4, jnp.ones((_C, 1), f32))
    bs4 = jnp.kron(eye4, jnp.ones((1, _C), f32))
    convr3 = jnp.transpose(conv_w.reshape(_NCLASSES, _C, _K), (2, 1, 0))
    conv4 = jax.vmap(lambda m: jnp.kron(eye4, m))(convr3)
    cb = jnp.tile(conv_b, _PPACK)[None]
    weights = (fc1bd, b1c, lnw, lnb, gmat, bg4, fc2bd, b2t, smat, bs4,
               conv4, cb)

    gxa, gpa = _sc_gather(idx2a, tx, tp)
    gxb, gpb = _sc_gather(idx2b, tx, tp)
    outa = _tc_mlp(gxa.reshape(_K, _QH, 128), gpa.reshape(_K, _QH, 128),
                   up128[:_QH], weights)
    outb = _tc_mlp(gxb.reshape(_K, _QH, 128), gpb.reshape(_K, _QH, 128),
                   up128[_QH:], weights)
    out = jnp.concatenate([outa, outb], axis=0)
    out = out.reshape(-1, _PPACK, _NCLASSES).transpose(2, 0, 1).reshape(
        _NCLASSES, _N)
    return out[None]


# speedup vs baseline: 4.3956x; 1.1950x over previous
import functools

import jax
import jax.numpy as jnp
from jax import lax
from jax.experimental import pallas as pl
from jax.experimental.pallas import tpu as pltpu
from jax.experimental.pallas import tpu_sc as plsc

_NCLASSES = 19
_S = 7
_K = _S * _S
_C = 32
_H, _W = 64, 2048
_PAD = (_S - 1) // 2
_HP = _H + 2 * _PAD
_WPP = 2064
_NPIX = _HP * _WPP
_N = 16384
_NH = _N // 2
_ROWSH = _K * _NH
_IDX_MINOR = 98
_CHUNK_GROUPS = 8
_CHUNK = _CHUNK_GROUPS * _IDX_MINOR
_NTILES = 32

_PPACK = 4
_QH = _NH // _PPACK
_QBLK = 64
_R = _K * _QBLK
_OL = _PPACK * _NCLASSES


_XCHUNK = 1032
_XIN = 1040
_UNITS = _HP * 2


def _sc_build_tables(x3, p3):
    mesh = plsc.VectorSubcoreMesh(core_axis_name="c", subcore_axis_name="s")

    @functools.partial(
        pl.kernel,
        mesh=mesh,
        compiler_params=pltpu.CompilerParams(use_tc_tiling_on_sc=False, needs_layout_passes=False),
        out_type=[
            jax.ShapeDtypeStruct((_NPIX, _C), jnp.float32),
            jax.ShapeDtypeStruct((_NPIX, _C), jnp.float32),
        ],
        scratch_types=[
            pltpu.VMEM((_C, _XIN), jnp.float32),
            pltpu.VMEM((4, _XIN), jnp.float32),
            pltpu.VMEM((_XCHUNK, _C), jnp.float32),
            pltpu.VMEM((_XCHUNK, _C), jnp.float32),
            pltpu.SemaphoreType.DMA,
        ],
    )
    def k(x_hbm, p_hbm, tx_hbm, tp_hbm, xbuf, pbuf, txbuf, tpbuf, sem):
        wid = lax.axis_index("s") * 2 + lax.axis_index("c")
        lanes = jax.lax.broadcasted_iota(jnp.int32, (16,), 0)
        zero16 = jnp.zeros((16,), jnp.float32)

        def zero_tp_hi(i, _):
            tpbuf[i, pl.ds(16, 16)] = zero16
            return _
        lax.fori_loop(0, _XCHUNK, zero_tp_hi, None)

        nunits = 4 + jnp.where(wid < _UNITS - 4 * _NTILES, 1, 0)

        def zero_row(ref):
            def zr(i, _):
                ref[i, pl.ds(0, 16)] = zero16
                ref[i, pl.ds(16, 16)] = zero16
                return _
            return zr

        def unit(t, _):
            u = wid + t * _NTILES
            y = u // 2
            xck = u % 2
            xc0 = xck * _XCHUNK
            p0 = y * _WPP + xc0
            interior = jnp.logical_and(y >= _PAD, y < _HP - _PAD)

            @pl.when(jnp.logical_not(interior))
            def _():
                lax.fori_loop(0, _XCHUNK, zero_row(txbuf), None)
                pltpu.sync_copy(txbuf, tx_hbm.at[pl.ds(p0, _XCHUNK)])
                pltpu.sync_copy(txbuf, tp_hbm.at[pl.ds(p0, _XCHUNK)])

            @pl.when(interior)
            def _():
                iy = y - _PAD
                in0 = pl.multiple_of(jnp.clip(((xc0 - _PAD) // 8) * 8, 0, _W - _XIN), 8)
                s = (xc0 - _PAD) - in0
                ilo = jnp.where(xck == 0, _PAD, 0)
                ihi = jnp.where(xck == 1, _W + _PAD - xc0, _XCHUNK)
                h1 = pltpu.async_copy(
                    x_hbm.at[:, iy, pl.ds(in0, _XIN)], xbuf, sem)
                h2 = pltpu.async_copy(
                    p_hbm.at[:, iy, pl.ds(in0, _XIN)], pbuf, sem)
                h1.wait()
                h2.wait()

                def edge_zero(i, _):
                    txbuf[i, pl.ds(0, 16)] = zero16
                    txbuf[i, pl.ds(16, 16)] = zero16
                    tpbuf[i, pl.ds(0, 16)] = zero16
                    return _

                lax.fori_loop(0, ilo, edge_zero, None)
                lax.fori_loop(ihi, _XCHUNK, edge_zero, None)

                def px(i, _):
                    col = i + s
                    cols = jnp.full((16,), col, jnp.int32)
                    v0 = plsc.load_gather(xbuf, [lanes, cols])
                    v1 = plsc.load_gather(xbuf, [lanes + 16, cols])
                    vp = plsc.load_gather(pbuf, [lanes & 3, cols])
                    txbuf[i, pl.ds(0, 16)] = v0
                    txbuf[i, pl.ds(16, 16)] = v1
                    tpbuf[i, pl.ds(0, 16)] = jnp.where(lanes < 4, vp, 0.0)
                    return _

                lax.fori_loop(ilo, ihi, px, None)
                pltpu.sync_copy(txbuf, tx_hbm.at[pl.ds(p0, _XCHUNK)])
                pltpu.sync_copy(tpbuf, tp_hbm.at[pl.ds(p0, _XCHUNK)])
            return _

        lax.fori_loop(0, nunits, unit, None)

    return k(x3, p3)


def _sc_gather(idx2, tx, tp):
    rows = idx2.shape[0] * _IDX_MINOR
    rows_per_tile = rows // _NTILES
    chunks_per_tile = rows_per_tile // _CHUNK
    mesh = plsc.VectorSubcoreMesh(core_axis_name="c", subcore_axis_name="s")

    @functools.partial(
        pl.kernel,
        mesh=mesh,
        compiler_params=pltpu.CompilerParams(use_tc_tiling_on_sc=False),
        out_type=[
            jax.ShapeDtypeStruct((rows, _C), jnp.float32),
            jax.ShapeDtypeStruct((rows, _C), jnp.float32),
        ],
        scratch_types=[
            pltpu.VMEM((_CHUNK_GROUPS, _IDX_MINOR), jnp.int32),
            pltpu.VMEM((_CHUNK, _C), jnp.float32),
            pltpu.VMEM((_CHUNK, _C), jnp.float32),
            pltpu.SemaphoreType.DMA,
        ],
    )
    def k(idx_hbm, tx_hbm, tp_hbm, gx_hbm, gp_hbm, idx_v, gx_v, gp_v, sem):
        wid = lax.axis_index("s") * 2 + lax.axis_index("c")
        idx_row0 = wid * (rows_per_tile // _IDX_MINOR)
        row0 = wid * rows_per_tile

        def body(c, _):
            pltpu.sync_copy(
                idx_hbm.at[pl.ds(idx_row0 + c * _CHUNK_GROUPS, _CHUNK_GROUPS)],
                idx_v)
            handles = []
            for j in range(_CHUNK_GROUPS):
                handles.append(pltpu.async_copy(
                    tx_hbm.at[idx_v.at[j]],
                    gx_v.at[pl.ds(j * _IDX_MINOR, _IDX_MINOR)], sem))
                handles.append(pltpu.async_copy(
                    tp_hbm.at[idx_v.at[j]],
                    gp_v.at[pl.ds(j * _IDX_MINOR, _IDX_MINOR)], sem))
            for h in handles:
                h.wait()
            base = row0 + c * _CHUNK
            pltpu.sync_copy(gx_v, gx_hbm.at[pl.ds(base, _CHUNK)])
            pltpu.sync_copy(gp_v, gp_hbm.at[pl.ds(base, _CHUNK)])
            return _

        lax.fori_loop(0, chunks_per_tile, body, None)

    return k(idx2, tx, tp)


def _tc_body(gx_ref, gp_ref, up_ref, fc1_ref, b1_ref, lnw_ref, lnb_ref,
             g_ref, bg_ref, fc2_ref, b2_ref, s_ref, bs_ref, conv_ref, cb_ref,
             out_ref):
    gp3 = gp_ref[...]
    up = up_ref[...]
    d = jnp.abs(gp3 - up[None, :, :]).reshape(_R, 128)
    hc = jnp.dot(d, fc1_ref[...], preferred_element_type=jnp.float32) + b1_ref[...]
    vs = jnp.dot(hc * hc, g_ref[...], preferred_element_type=jnp.float32)
    inv = jnp.dot(lax.rsqrt(vs + 1e-5), bg_ref[...],
                  preferred_element_type=jnp.float32)
    h = hc * inv * lnw_ref[...] + lnb_ref[...]
    h = 0.5 * h * (1.0 + lax.erf(h * 0.7071067811865476))
    h = jnp.dot(h, fc2_ref[...], preferred_element_type=jnp.float32) + b2_ref[...]
    e = jnp.exp(h)
    s = jnp.dot(e, s_ref[...], preferred_element_type=jnp.float32)
    w = e * jnp.dot(1.0 / s, bs_ref[...], preferred_element_type=jnp.float32)
    wx = gx_ref[...].reshape(_R, 128) * w
    wx3 = wx.reshape(_K, _QBLK, 128)
    res = lax.dot_general(
        wx3, conv_ref[...],
        dimension_numbers=(((2,), (1,)), ((0,), (0,))),
        preferred_element_type=jnp.float32)
    out_ref[...] = jnp.sum(res, axis=0) + cb_ref[...]


def _tc_mlp(gx3, gp3, up128, weights):
    q = up128.shape[0]
    return pl.pallas_call(
        _tc_body,
        grid=(q // _QBLK,),
        in_specs=[
            pl.BlockSpec((_K, _QBLK, 128), lambda i: (0, i, 0)),
            pl.BlockSpec((_K, _QBLK, 128), lambda i: (0, i, 0)),
            pl.BlockSpec((_QBLK, 128), lambda i: (i, 0)),
            pl.BlockSpec((128, 256), lambda i: (0, 0)),
            pl.BlockSpec((256,), lambda i: (0,)),
            pl.BlockSpec((256,), lambda i: (0,)),
            pl.BlockSpec((256,), lambda i: (0,)),
            pl.BlockSpec((256, _PPACK), lambda i: (0, 0)),
            pl.BlockSpec((_PPACK, 256), lambda i: (0, 0)),
            pl.BlockSpec((256, 128), lambda i: (0, 0)),
            pl.BlockSpec((128,), lambda i: (0,)),
            pl.BlockSpec((128, _PPACK), lambda i: (0, 0)),
            pl.BlockSpec((_PPACK, 128), lambda i: (0, 0)),
            pl.BlockSpec((_K, 128, _OL), lambda i: (0, 0, 0)),
            pl.BlockSpec((1, _OL), lambda i: (0, 0)),
        ],
        out_specs=pl.BlockSpec((_QBLK, _OL), lambda i: (i, 0)),
        out_shape=jax.ShapeDtypeStruct((q, _OL), jnp.float32),
    )(gx3, gp3, up128, *weights)


def kernel(x, proj_range_xyz, unproj_range_xyz, p2ri_lut, num_valid_pts,
           fc1_w, fc1_b, ln_w, ln_b, fc2_w, fc2_b, conv_w, conv_b):
    f32 = jnp.float32
    tx, tp = _sc_build_tables(x[0], proj_range_xyz[0])

    lut = p2ri_lut[0]
    yc = lut[:, 1]
    xc = lut[:, 2]
    dy = jnp.arange(_S, dtype=jnp.int32)
    off = (dy[:, None] * _WPP + dy[None, :]).reshape(_K)
    base = yc * _WPP + xc
    idx = off[:, None] + base[None, :]
    idx2a = idx[:, :_NH].reshape(-1, _IDX_MINOR).astype(jnp.int32)
    idx2b = idx[:, _NH:].reshape(-1, _IDX_MINOR).astype(jnp.int32)

    up128 = jnp.pad(unproj_range_xyz[0], ((0, 0), (0, _C - 4))).reshape(-1, 128)

    eye4 = jnp.eye(_PPACK, dtype=f32)
    fc1p = jnp.pad(fc1_w.T, ((0, _C - 4), (0, 0)))
    fc1c = fc1p - jnp.mean(fc1p, axis=1, keepdims=True)
    fc1bd = jnp.kron(eye4, fc1c)
    b1c = jnp.tile(fc1_b - jnp.mean(fc1_b), _PPACK)
    lnw = jnp.tile(ln_w, _PPACK)
    lnb = jnp.tile(ln_b, _PPACK)
    gmat = jnp.kron(eye4, jnp.full((64, 1), 1.0 / 64, f32))
    bg4 = jnp.kron(eye4, jnp.ones((1, 64), f32))
    fc2bd = jnp.kron(eye4, fc2_w.T)
    b2t = jnp.tile(fc2_b, _PPACK)
    smat = jnp.kron(eye4, jnp.ones((_C, 1), f32))
    bs4 = jnp.kron(eye4, jnp.ones((1, _C), f32))
    convr3 = jnp.transpose(conv_w.reshape(_NCLASSES, _C, _K), (2, 1, 0))
    conv4 = jax.vmap(lambda m: jnp.kron(eye4, m))(convr3)
    cb = jnp.tile(conv_b, _PPACK)[None]
    weights = (fc1bd, b1c, lnw, lnb, gmat, bg4, fc2bd, b2t, smat, bs4,
               conv4, cb)

    gxa, gpa = _sc_gather(idx2a, tx, tp)
    gxb, gpb = _sc_gather(idx2b, tx, tp)
    outa = _tc_mlp(gxa.reshape(_K, _QH, 128), gpa.reshape(_K, _QH, 128),
                   up128[:_QH], weights)
    outb = _tc_mlp(gxb.reshape(_K, _QH, 128), gpb.reshape(_K, _QH, 128),
                   up128[_QH:], weights)
    out = jnp.concatenate([outa, outb], axis=0)
    out = out.reshape(-1, _PPACK, _NCLASSES).transpose(2, 0, 1).reshape(
        _NCLASSES, _N)
    return out[None]


# speedup vs baseline: 4.4864x; 1.0206x over previous
import functools

import jax
import jax.numpy as jnp
from jax import lax
from jax.experimental import pallas as pl
from jax.experimental.pallas import tpu as pltpu
from jax.experimental.pallas import tpu_sc as plsc

_NCLASSES = 19
_S = 7
_K = _S * _S
_C = 32
_H, _W = 64, 2048
_PAD = (_S - 1) // 2
_HP = _H + 2 * _PAD
_WPP = 2064
_NPIX = _HP * _WPP
_N = 16384
_NH = _N // 2
_ROWSH = _K * _NH
_IDX_MINOR = 98
_CHUNK_GROUPS = 8
_CHUNK = _CHUNK_GROUPS * _IDX_MINOR
_NTILES = 32

_PPACK = 4
_QH = _NH // _PPACK
_QBLK = 128
_R = _K * _QBLK
_OL = _PPACK * _NCLASSES


_XCHUNK = 1032
_XIN = 1040
_UNITS = _HP * 2


def _sc_build_tables(x3, p3):
    mesh = plsc.VectorSubcoreMesh(core_axis_name="c", subcore_axis_name="s")

    @functools.partial(
        pl.kernel,
        mesh=mesh,
        compiler_params=pltpu.CompilerParams(use_tc_tiling_on_sc=False, needs_layout_passes=False),
        out_type=[
            jax.ShapeDtypeStruct((_NPIX, _C), jnp.float32),
            jax.ShapeDtypeStruct((_NPIX, _C), jnp.float32),
        ],
        scratch_types=[
            pltpu.VMEM((_C, _XIN), jnp.float32),
            pltpu.VMEM((4, _XIN), jnp.float32),
            pltpu.VMEM((_XCHUNK, _C), jnp.float32),
            pltpu.VMEM((_XCHUNK, _C), jnp.float32),
            pltpu.SemaphoreType.DMA,
        ],
    )
    def k(x_hbm, p_hbm, tx_hbm, tp_hbm, xbuf, pbuf, txbuf, tpbuf, sem):
        wid = lax.axis_index("s") * 2 + lax.axis_index("c")
        lanes = jax.lax.broadcasted_iota(jnp.int32, (16,), 0)
        zero16 = jnp.zeros((16,), jnp.float32)

        def zero_tp_hi(i, _):
            tpbuf[i, pl.ds(16, 16)] = zero16
            return _
        lax.fori_loop(0, _XCHUNK, zero_tp_hi, None)

        nunits = 4 + jnp.where(wid < _UNITS - 4 * _NTILES, 1, 0)

        def zero_row(ref):
            def zr(i, _):
                ref[i, pl.ds(0, 16)] = zero16
                ref[i, pl.ds(16, 16)] = zero16
                return _
            return zr

        def unit(t, _):
            u = wid + t * _NTILES
            y = u // 2
            xck = u % 2
            xc0 = xck * _XCHUNK
            p0 = y * _WPP + xc0
            interior = jnp.logical_and(y >= _PAD, y < _HP - _PAD)

            @pl.when(jnp.logical_not(interior))
            def _():
                lax.fori_loop(0, _XCHUNK, zero_row(txbuf), None)
                pltpu.sync_copy(txbuf, tx_hbm.at[pl.ds(p0, _XCHUNK)])
                pltpu.sync_copy(txbuf, tp_hbm.at[pl.ds(p0, _XCHUNK)])

            @pl.when(interior)
            def _():
                iy = y - _PAD
                in0 = pl.multiple_of(jnp.clip(((xc0 - _PAD) // 8) * 8, 0, _W - _XIN), 8)
                s = (xc0 - _PAD) - in0
                ilo = jnp.where(xck == 0, _PAD, 0)
                ihi = jnp.where(xck == 1, _W + _PAD - xc0, _XCHUNK)
                h1 = pltpu.async_copy(
                    x_hbm.at[:, iy, pl.ds(in0, _XIN)], xbuf, sem)
                h2 = pltpu.async_copy(
                    p_hbm.at[:, iy, pl.ds(in0, _XIN)], pbuf, sem)
                h1.wait()
                h2.wait()

                def edge_zero(i, _):
                    txbuf[i, pl.ds(0, 16)] = zero16
                    txbuf[i, pl.ds(16, 16)] = zero16
                    tpbuf[i, pl.ds(0, 16)] = zero16
                    return _

                lax.fori_loop(0, ilo, edge_zero, None)
                lax.fori_loop(ihi, _XCHUNK, edge_zero, None)

                def px(i, _):
                    col = i + s
                    cols = jnp.full((16,), col, jnp.int32)
                    v0 = plsc.load_gather(xbuf, [lanes, cols])
                    v1 = plsc.load_gather(xbuf, [lanes + 16, cols])
                    vp = plsc.load_gather(pbuf, [lanes & 3, cols])
                    txbuf[i, pl.ds(0, 16)] = v0
                    txbuf[i, pl.ds(16, 16)] = v1
                    tpbuf[i, pl.ds(0, 16)] = jnp.where(lanes < 4, vp, 0.0)
                    return _

                lax.fori_loop(ilo, ihi, px, None)
                pltpu.sync_copy(txbuf, tx_hbm.at[pl.ds(p0, _XCHUNK)])
                pltpu.sync_copy(tpbuf, tp_hbm.at[pl.ds(p0, _XCHUNK)])
            return _

        lax.fori_loop(0, nunits, unit, None)

    return k(x3, p3)


def _sc_gather(idx2, tx, tp):
    rows = idx2.shape[0] * _IDX_MINOR
    rows_per_tile = rows // _NTILES
    chunks_per_tile = rows_per_tile // _CHUNK
    mesh = plsc.VectorSubcoreMesh(core_axis_name="c", subcore_axis_name="s")

    @functools.partial(
        pl.kernel,
        mesh=mesh,
        compiler_params=pltpu.CompilerParams(use_tc_tiling_on_sc=False),
        out_type=[
            jax.ShapeDtypeStruct((rows, _C), jnp.float32),
            jax.ShapeDtypeStruct((rows, _C), jnp.float32),
        ],
        scratch_types=[
            pltpu.VMEM((_CHUNK_GROUPS, _IDX_MINOR), jnp.int32),
            pltpu.VMEM((_CHUNK, _C), jnp.float32),
            pltpu.VMEM((_CHUNK, _C), jnp.float32),
            pltpu.SemaphoreType.DMA,
        ],
    )
    def k(idx_hbm, tx_hbm, tp_hbm, gx_hbm, gp_hbm, idx_v, gx_v, gp_v, sem):
        wid = lax.axis_index("s") * 2 + lax.axis_index("c")
        idx_row0 = wid * (rows_per_tile // _IDX_MINOR)
        row0 = wid * rows_per_tile

        def body(c, _):
            pltpu.sync_copy(
                idx_hbm.at[pl.ds(idx_row0 + c * _CHUNK_GROUPS, _CHUNK_GROUPS)],
                idx_v)
            handles = []
            for j in range(_CHUNK_GROUPS):
                handles.append(pltpu.async_copy(
                    tx_hbm.at[idx_v.at[j]],
                    gx_v.at[pl.ds(j * _IDX_MINOR, _IDX_MINOR)], sem))
                handles.append(pltpu.async_copy(
                    tp_hbm.at[idx_v.at[j]],
                    gp_v.at[pl.ds(j * _IDX_MINOR, _IDX_MINOR)], sem))
            for h in handles:
                h.wait()
            base = row0 + c * _CHUNK
            pltpu.sync_copy(gx_v, gx_hbm.at[pl.ds(base, _CHUNK)])
            pltpu.sync_copy(gp_v, gp_hbm.at[pl.ds(base, _CHUNK)])
            return _

        lax.fori_loop(0, chunks_per_tile, body, None)

    return k(idx2, tx, tp)


def _tc_body(gx_ref, gp_ref, up_ref, fc1_ref, b1_ref, lnw_ref, lnb_ref,
             g_ref, bg_ref, fc2_ref, b2_ref, s_ref, bs_ref, conv_ref, cb_ref,
             out_ref):
    gp3 = gp_ref[...]
    up = up_ref[...]
    d = jnp.abs(gp3 - up[None, :, :]).reshape(_R, 128)
    hc = jnp.dot(d, fc1_ref[...], preferred_element_type=jnp.float32) + b1_ref[...]
    vs = jnp.dot(hc * hc, g_ref[...], preferred_element_type=jnp.float32)
    inv = jnp.dot(lax.rsqrt(vs + 1e-5), bg_ref[...],
                  preferred_element_type=jnp.float32)
    h = hc * inv * lnw_ref[...] + lnb_ref[...]
    h = 0.5 * h * (1.0 + lax.erf(h * 0.7071067811865476))
    h = jnp.dot(h, fc2_ref[...], preferred_element_type=jnp.float32) + b2_ref[...]
    e = jnp.exp(h)
    s = jnp.dot(e, s_ref[...], preferred_element_type=jnp.float32)
    w = e * jnp.dot(1.0 / s, bs_ref[...], preferred_element_type=jnp.float32)
    wx = gx_ref[...].reshape(_R, 128) * w
    wx3 = wx.reshape(_K, _QBLK, 128)
    res = lax.dot_general(
        wx3, conv_ref[...],
        dimension_numbers=(((2,), (1,)), ((0,), (0,))),
        preferred_element_type=jnp.float32)
    out_ref[...] = jnp.sum(res, axis=0) + cb_ref[...]


def _tc_mlp(gx3, gp3, up128, weights):
    q = up128.shape[0]
    return pl.pallas_call(
        _tc_body,
        grid=(q // _QBLK,),
        in_specs=[
            pl.BlockSpec((_K, _QBLK, 128), lambda i: (0, i, 0)),
            pl.BlockSpec((_K, _QBLK, 128), lambda i: (0, i, 0)),
            pl.BlockSpec((_QBLK, 128), lambda i: (i, 0)),
            pl.BlockSpec((128, 256), lambda i: (0, 0)),
            pl.BlockSpec((256,), lambda i: (0,)),
            pl.BlockSpec((256,), lambda i: (0,)),
            pl.BlockSpec((256,), lambda i: (0,)),
            pl.BlockSpec((256, _PPACK), lambda i: (0, 0)),
            pl.BlockSpec((_PPACK, 256), lambda i: (0, 0)),
            pl.BlockSpec((256, 128), lambda i: (0, 0)),
            pl.BlockSpec((128,), lambda i: (0,)),
            pl.BlockSpec((128, _PPACK), lambda i: (0, 0)),
            pl.BlockSpec((_PPACK, 128), lambda i: (0, 0)),
            pl.BlockSpec((_K, 128, _OL), lambda i: (0, 0, 0)),
            pl.BlockSpec((1, _OL), lambda i: (0, 0)),
        ],
        out_specs=pl.BlockSpec((_QBLK, _OL), lambda i: (i, 0)),
        out_shape=jax.ShapeDtypeStruct((q, _OL), jnp.float32),
    )(gx3, gp3, up128, *weights)


def kernel(x, proj_range_xyz, unproj_range_xyz, p2ri_lut, num_valid_pts,
           fc1_w, fc1_b, ln_w, ln_b, fc2_w, fc2_b, conv_w, conv_b):
    f32 = jnp.float32
    tx, tp = _sc_build_tables(x[0], proj_range_xyz[0])

    lut = p2ri_lut[0]
    yc = lut[:, 1]
    xc = lut[:, 2]
    dy = jnp.arange(_S, dtype=jnp.int32)
    off = (dy[:, None] * _WPP + dy[None, :]).reshape(_K)
    base = yc * _WPP + xc
    idx = off[:, None] + base[None, :]
    idx2a = idx[:, :_NH].reshape(-1, _IDX_MINOR).astype(jnp.int32)
    idx2b = idx[:, _NH:].reshape(-1, _IDX_MINOR).astype(jnp.int32)

    up128 = jnp.pad(unproj_range_xyz[0], ((0, 0), (0, _C - 4))).reshape(-1, 128)

    eye4 = jnp.eye(_PPACK, dtype=f32)
    fc1p = jnp.pad(fc1_w.T, ((0, _C - 4), (0, 0)))
    fc1c = fc1p - jnp.mean(fc1p, axis=1, keepdims=True)
    fc1bd = jnp.kron(eye4, fc1c)
    b1c = jnp.tile(fc1_b - jnp.mean(fc1_b), _PPACK)
    lnw = jnp.tile(ln_w, _PPACK)
    lnb = jnp.tile(ln_b, _PPACK)
    gmat = jnp.kron(eye4, jnp.full((64, 1), 1.0 / 64, f32))
    bg4 = jnp.kron(eye4, jnp.ones((1, 64), f32))
    fc2bd = jnp.kron(eye4, fc2_w.T)
    b2t = jnp.tile(fc2_b, _PPACK)
    smat = jnp.kron(eye4, jnp.ones((_C, 1), f32))
    bs4 = jnp.kron(eye4, jnp.ones((1, _C), f32))
    convr3 = jnp.transpose(conv_w.reshape(_NCLASSES, _C, _K), (2, 1, 0))
    conv4 = jax.vmap(lambda m: jnp.kron(eye4, m))(convr3)
    cb = jnp.tile(conv_b, _PPACK)[None]
    weights = (fc1bd, b1c, lnw, lnb, gmat, bg4, fc2bd, b2t, smat, bs4,
               conv4, cb)

    gxa, gpa = _sc_gather(idx2a, tx, tp)
    gxb, gpb = _sc_gather(idx2b, tx, tp)
    outa = _tc_mlp(gxa.reshape(_K, _QH, 128), gpa.reshape(_K, _QH, 128),
                   up128[:_QH], weights)
    outb = _tc_mlp(gxb.reshape(_K, _QH, 128), gpb.reshape(_K, _QH, 128),
                   up128[_QH:], weights)
    out = jnp.concatenate([outa, outb], axis=0)
    out = out.reshape(-1, _PPACK, _NCLASSES).transpose(2, 0, 1).reshape(
        _NCLASSES, _N)
    return out[None]


# speedup vs baseline: 4.4914x; 1.0011x over previous
import functools

import jax
import jax.numpy as jnp
from jax import lax
from jax.experimental import pallas as pl
from jax.experimental.pallas import tpu as pltpu
from jax.experimental.pallas import tpu_sc as plsc

_NCLASSES = 19
_S = 7
_K = _S * _S
_C = 32
_H, _W = 64, 2048
_PAD = (_S - 1) // 2
_HP = _H + 2 * _PAD
_WPP = 2064
_NPIX = _HP * _WPP
_N = 16384
_NH = _N // 2
_ROWSH = _K * _NH
_IDX_MINOR = 98
_CHUNK_GROUPS = 8
_CHUNK = _CHUNK_GROUPS * _IDX_MINOR
_NTILES = 32

_PPACK = 4
_QH = _NH // _PPACK
_QBLK = 128
_R = _K * _QBLK
_OL = _PPACK * _NCLASSES


_XCHUNK = 1032
_XIN = 1040
_UNITS = _HP * 2


def _sc_build_tables(x3, p3):
    mesh = plsc.VectorSubcoreMesh(core_axis_name="c", subcore_axis_name="s")

    @functools.partial(
        pl.kernel,
        mesh=mesh,
        compiler_params=pltpu.CompilerParams(use_tc_tiling_on_sc=False, needs_layout_passes=False),
        out_type=[
            jax.ShapeDtypeStruct((_NPIX, _C), jnp.float32),
            jax.ShapeDtypeStruct((_NPIX, _C), jnp.float32),
        ],
        scratch_types=[
            pltpu.VMEM((_C, _XIN), jnp.float32),
            pltpu.VMEM((4, _XIN), jnp.float32),
            pltpu.VMEM((_XCHUNK, _C), jnp.float32),
            pltpu.VMEM((_XCHUNK, _C), jnp.float32),
            pltpu.SemaphoreType.DMA,
        ],
    )
    def k(x_hbm, p_hbm, tx_hbm, tp_hbm, xbuf, pbuf, txbuf, tpbuf, sem):
        wid = lax.axis_index("s") * 2 + lax.axis_index("c")
        lanes = jax.lax.broadcasted_iota(jnp.int32, (16,), 0)
        zero16 = jnp.zeros((16,), jnp.float32)

        def zero_tp_hi(i, _):
            tpbuf[i, pl.ds(16, 16)] = zero16
            return _
        lax.fori_loop(0, _XCHUNK, zero_tp_hi, None)

        nunits = 4 + jnp.where(wid < _UNITS - 4 * _NTILES, 1, 0)

        def zero_row(ref):
            def zr(i, _):
                ref[i, pl.ds(0, 16)] = zero16
                ref[i, pl.ds(16, 16)] = zero16
                return _
            return zr

        def unit(t, _):
            u = wid + t * _NTILES
            y = u // 2
            xck = u % 2
            xc0 = xck * _XCHUNK
            p0 = y * _WPP + xc0
            interior = jnp.logical_and(y >= _PAD, y < _HP - _PAD)

            @pl.when(jnp.logical_not(interior))
            def _():
                lax.fori_loop(0, _XCHUNK, zero_row(txbuf), None)
                pltpu.sync_copy(txbuf, tx_hbm.at[pl.ds(p0, _XCHUNK)])
                pltpu.sync_copy(txbuf, tp_hbm.at[pl.ds(p0, _XCHUNK)])

            @pl.when(interior)
            def _():
                iy = y - _PAD
                in0 = pl.multiple_of(jnp.clip(((xc0 - _PAD) // 8) * 8, 0, _W - _XIN), 8)
                s = (xc0 - _PAD) - in0
                ilo = jnp.where(xck == 0, _PAD, 0)
                ihi = jnp.where(xck == 1, _W + _PAD - xc0, _XCHUNK)
                h1 = pltpu.async_copy(
                    x_hbm.at[:, iy, pl.ds(in0, _XIN)], xbuf, sem)
                h2 = pltpu.async_copy(
                    p_hbm.at[:, iy, pl.ds(in0, _XIN)], pbuf, sem)
                h1.wait()
                h2.wait()

                def edge_zero(i, _):
                    txbuf[i, pl.ds(0, 16)] = zero16
                    txbuf[i, pl.ds(16, 16)] = zero16
                    tpbuf[i, pl.ds(0, 16)] = zero16
                    return _

                def px4(t, _):
                    for j in range(4):
                        i = t * 4 + j
                        col = jnp.clip(i + s, 0, _XIN - 1)
                        cols = jnp.full((16,), col, jnp.int32)
                        v0 = plsc.load_gather(xbuf, [lanes, cols])
                        v1 = plsc.load_gather(xbuf, [lanes + 16, cols])
                        vp = plsc.load_gather(pbuf, [lanes & 3, cols])
                        txbuf[i, pl.ds(0, 16)] = v0
                        txbuf[i, pl.ds(16, 16)] = v1
                        tpbuf[i, pl.ds(0, 16)] = jnp.where(lanes < 4, vp, 0.0)
                    return _

                lax.fori_loop(0, _XCHUNK // 4, px4, None)
                lax.fori_loop(0, ilo, edge_zero, None)
                lax.fori_loop(ihi, _XCHUNK, edge_zero, None)
                pltpu.sync_copy(txbuf, tx_hbm.at[pl.ds(p0, _XCHUNK)])
                pltpu.sync_copy(tpbuf, tp_hbm.at[pl.ds(p0, _XCHUNK)])
            return _

        lax.fori_loop(0, nunits, unit, None)

    return k(x3, p3)


def _sc_gather(idx2, tx, tp):
    rows = idx2.shape[0] * _IDX_MINOR
    rows_per_tile = rows // _NTILES
    chunks_per_tile = rows_per_tile // _CHUNK
    mesh = plsc.VectorSubcoreMesh(core_axis_name="c", subcore_axis_name="s")

    @functools.partial(
        pl.kernel,
        mesh=mesh,
        compiler_params=pltpu.CompilerParams(use_tc_tiling_on_sc=False),
        out_type=[
            jax.ShapeDtypeStruct((rows, _C), jnp.float32),
            jax.ShapeDtypeStruct((rows, _C), jnp.float32),
        ],
        scratch_types=[
            pltpu.VMEM((_CHUNK_GROUPS, _IDX_MINOR), jnp.int32),
            pltpu.VMEM((_CHUNK, _C), jnp.float32),
            pltpu.VMEM((_CHUNK, _C), jnp.float32),
            pltpu.SemaphoreType.DMA,
        ],
    )
    def k(idx_hbm, tx_hbm, tp_hbm, gx_hbm, gp_hbm, idx_v, gx_v, gp_v, sem):
        wid = lax.axis_index("s") * 2 + lax.axis_index("c")
        idx_row0 = wid * (rows_per_tile // _IDX_MINOR)
        row0 = wid * rows_per_tile

        def body(c, _):
            pltpu.sync_copy(
                idx_hbm.at[pl.ds(idx_row0 + c * _CHUNK_GROUPS, _CHUNK_GROUPS)],
                idx_v)
            handles = []
            for j in range(_CHUNK_GROUPS):
                handles.append(pltpu.async_copy(
                    tx_hbm.at[idx_v.at[j]],
                    gx_v.at[pl.ds(j * _IDX_MINOR, _IDX_MINOR)], sem))
                handles.append(pltpu.async_copy(
                    tp_hbm.at[idx_v.at[j]],
                    gp_v.at[pl.ds(j * _IDX_MINOR, _IDX_MINOR)], sem))
            for h in handles:
                h.wait()
            base = row0 + c * _CHUNK
            pltpu.sync_copy(gx_v, gx_hbm.at[pl.ds(base, _CHUNK)])
            pltpu.sync_copy(gp_v, gp_hbm.at[pl.ds(base, _CHUNK)])
            return _

        lax.fori_loop(0, chunks_per_tile, body, None)

    return k(idx2, tx, tp)


def _tc_body(gx_ref, gp_ref, up_ref, fc1_ref, b1_ref, lnw_ref, lnb_ref,
             g_ref, bg_ref, fc2_ref, b2_ref, s_ref, bs_ref, conv_ref, cb_ref,
             out_ref):
    gp3 = gp_ref[...]
    up = up_ref[...]
    d = jnp.abs(gp3 - up[None, :, :]).reshape(_R, 128)
    hc = jnp.dot(d, fc1_ref[...], preferred_element_type=jnp.float32) + b1_ref[...]
    vs = jnp.dot(hc * hc, g_ref[...], preferred_element_type=jnp.float32)
    inv = jnp.dot(lax.rsqrt(vs + 1e-5), bg_ref[...],
                  preferred_element_type=jnp.float32)
    h = hc * inv * lnw_ref[...] + lnb_ref[...]
    h = 0.5 * h * (1.0 + lax.erf(h * 0.7071067811865476))
    h = jnp.dot(h, fc2_ref[...], preferred_element_type=jnp.float32) + b2_ref[...]
    e = jnp.exp(h)
    s = jnp.dot(e, s_ref[...], preferred_element_type=jnp.float32)
    w = e * jnp.dot(1.0 / s, bs_ref[...], preferred_element_type=jnp.float32)
    wx = gx_ref[...].reshape(_R, 128) * w
    wx3 = wx.reshape(_K, _QBLK, 128)
    res = lax.dot_general(
        wx3, conv_ref[...],
        dimension_numbers=(((2,), (1,)), ((0,), (0,))),
        preferred_element_type=jnp.float32)
    out_ref[...] = jnp.sum(res, axis=0) + cb_ref[...]


def _tc_mlp(gx3, gp3, up128, weights):
    q = up128.shape[0]
    return pl.pallas_call(
        _tc_body,
        grid=(q // _QBLK,),
        in_specs=[
            pl.BlockSpec((_K, _QBLK, 128), lambda i: (0, i, 0)),
            pl.BlockSpec((_K, _QBLK, 128), lambda i: (0, i, 0)),
            pl.BlockSpec((_QBLK, 128), lambda i: (i, 0)),
            pl.BlockSpec((128, 256), lambda i: (0, 0)),
            pl.BlockSpec((256,), lambda i: (0,)),
            pl.BlockSpec((256,), lambda i: (0,)),
            pl.BlockSpec((256,), lambda i: (0,)),
            pl.BlockSpec((256, _PPACK), lambda i: (0, 0)),
            pl.BlockSpec((_PPACK, 256), lambda i: (0, 0)),
            pl.BlockSpec((256, 128), lambda i: (0, 0)),
            pl.BlockSpec((128,), lambda i: (0,)),
            pl.BlockSpec((128, _PPACK), lambda i: (0, 0)),
            pl.BlockSpec((_PPACK, 128), lambda i: (0, 0)),
            pl.BlockSpec((_K, 128, _OL), lambda i: (0, 0, 0)),
            pl.BlockSpec((1, _OL), lambda i: (0, 0)),
        ],
        out_specs=pl.BlockSpec((_QBLK, _OL), lambda i: (i, 0)),
        out_shape=jax.ShapeDtypeStruct((q, _OL), jnp.float32),
    )(gx3, gp3, up128, *weights)


def kernel(x, proj_range_xyz, unproj_range_xyz, p2ri_lut, num_valid_pts,
           fc1_w, fc1_b, ln_w, ln_b, fc2_w, fc2_b, conv_w, conv_b):
    f32 = jnp.float32
    tx, tp = _sc_build_tables(x[0], proj_range_xyz[0])

    lut = p2ri_lut[0]
    yc = lut[:, 1]
    xc = lut[:, 2]
    dy = jnp.arange(_S, dtype=jnp.int32)
    off = (dy[:, None] * _WPP + dy[None, :]).reshape(_K)
    base = yc * _WPP + xc
    idx = off[:, None] + base[None, :]
    idx2a = idx[:, :_NH].reshape(-1, _IDX_MINOR).astype(jnp.int32)
    idx2b = idx[:, _NH:].reshape(-1, _IDX_MINOR).astype(jnp.int32)

    up128 = jnp.pad(unproj_range_xyz[0], ((0, 0), (0, _C - 4))).reshape(-1, 128)

    eye4 = jnp.eye(_PPACK, dtype=f32)
    fc1p = jnp.pad(fc1_w.T, ((0, _C - 4), (0, 0)))
    fc1c = fc1p - jnp.mean(fc1p, axis=1, keepdims=True)
    fc1bd = jnp.kron(eye4, fc1c)
    b1c = jnp.tile(fc1_b - jnp.mean(fc1_b), _PPACK)
    lnw = jnp.tile(ln_w, _PPACK)
    lnb = jnp.tile(ln_b, _PPACK)
    gmat = jnp.kron(eye4, jnp.full((64, 1), 1.0 / 64, f32))
    bg4 = jnp.kron(eye4, jnp.ones((1, 64), f32))
    fc2bd = jnp.kron(eye4, fc2_w.T)
    b2t = jnp.tile(fc2_b, _PPACK)
    smat = jnp.kron(eye4, jnp.ones((_C, 1), f32))
    bs4 = jnp.kron(eye4, jnp.ones((1, _C), f32))
    convr3 = jnp.transpose(conv_w.reshape(_NCLASSES, _C, _K), (2, 1, 0))
    conv4 = jax.vmap(lambda m: jnp.kron(eye4, m))(convr3)
    cb = jnp.tile(conv_b, _PPACK)[None]
    weights = (fc1bd, b1c, lnw, lnb, gmat, bg4, fc2bd, b2t, smat, bs4,
               conv4, cb)

    gxa, gpa = _sc_gather(idx2a, tx, tp)
    gxb, gpb = _sc_gather(idx2b, tx, tp)
    outa = _tc_mlp(gxa.reshape(_K, _QH, 128), gpa.reshape(_K, _QH, 128),
                   up128[:_QH], weights)
    outb = _tc_mlp(gxb.reshape(_K, _QH, 128), gpb.reshape(_K, _QH, 128),
                   up128[_QH:], weights)
    out = jnp.concatenate([outa, outb], axis=0)
    out = out.reshape(-1, _PPACK, _NCLASSES).transpose(2, 0, 1).reshape(
        _NCLASSES, _N)
    return out[None]


# speedup vs baseline: 4.6345x; 1.0319x over previous
import functools

import jax
import jax.numpy as jnp
from jax import lax
from jax.experimental import pallas as pl
from jax.experimental.pallas import tpu as pltpu
from jax.experimental.pallas import tpu_sc as plsc

_NCLASSES = 19
_S = 7
_K = _S * _S
_C = 32
_H, _W = 64, 2048
_PAD = (_S - 1) // 2
_HP = _H + 2 * _PAD
_WPP = 2064
_NPIX = _HP * _WPP
_N = 16384
_NH = _N // 2
_ROWSH = _K * _NH
_IDX_MINOR = 98
_CHUNK_GROUPS = 8
_CHUNK = _CHUNK_GROUPS * _IDX_MINOR
_NTILES = 32

_PPACK = 4
_QH = _NH // _PPACK
_QBLK = 128
_R = _K * _QBLK
_OL = _PPACK * _NCLASSES


_XCHUNK = 1032
_XIN = 1040
_UNITS = _HP * 2


def _sc_build_tables(x3, p3):
    mesh = plsc.VectorSubcoreMesh(core_axis_name="c", subcore_axis_name="s")

    @functools.partial(
        pl.kernel,
        mesh=mesh,
        compiler_params=pltpu.CompilerParams(use_tc_tiling_on_sc=False, needs_layout_passes=False),
        out_type=[
            jax.ShapeDtypeStruct((_NPIX, _C), jnp.float32),
            jax.ShapeDtypeStruct((_NPIX, _C), jnp.float32),
        ],
        scratch_types=[
            pltpu.VMEM((_C, _XIN), jnp.float32),
            pltpu.VMEM((4, _XIN), jnp.float32),
            pltpu.VMEM((_XCHUNK, _C), jnp.float32),
            pltpu.VMEM((_XCHUNK, _C), jnp.float32),
            pltpu.SemaphoreType.DMA,
        ],
    )
    def k(x_hbm, p_hbm, tx_hbm, tp_hbm, xbuf, pbuf, txbuf, tpbuf, sem):
        wid = lax.axis_index("s") * 2 + lax.axis_index("c")
        lanes = jax.lax.broadcasted_iota(jnp.int32, (16,), 0)
        zero16 = jnp.zeros((16,), jnp.float32)

        def zero_tp_hi(i, _):
            tpbuf[i, pl.ds(16, 16)] = zero16
            return _
        lax.fori_loop(0, _XCHUNK, zero_tp_hi, None)

        nunits = 4 + jnp.where(wid < _UNITS - 4 * _NTILES, 1, 0)

        def zero_row(ref):
            def zr(i, _):
                ref[i, pl.ds(0, 16)] = zero16
                ref[i, pl.ds(16, 16)] = zero16
                return _
            return zr

        def unit(t, _):
            u = wid + t * _NTILES
            y = u // 2
            xck = u % 2
            xc0 = xck * _XCHUNK
            p0 = y * _WPP + xc0
            interior = jnp.logical_and(y >= _PAD, y < _HP - _PAD)

            @pl.when(jnp.logical_not(interior))
            def _():
                lax.fori_loop(0, _XCHUNK, zero_row(txbuf), None)
                pltpu.sync_copy(txbuf, tx_hbm.at[pl.ds(p0, _XCHUNK)])
                pltpu.sync_copy(txbuf, tp_hbm.at[pl.ds(p0, _XCHUNK)])

            @pl.when(interior)
            def _():
                iy = y - _PAD
                in0 = pl.multiple_of(jnp.clip(((xc0 - _PAD) // 8) * 8, 0, _W - _XIN), 8)
                s = (xc0 - _PAD) - in0
                ilo = jnp.where(xck == 0, _PAD, 0)
                ihi = jnp.where(xck == 1, _W + _PAD - xc0, _XCHUNK)
                h1 = pltpu.async_copy(
                    x_hbm.at[:, iy, pl.ds(in0, _XIN)], xbuf, sem)
                h2 = pltpu.async_copy(
                    p_hbm.at[:, iy, pl.ds(in0, _XIN)], pbuf, sem)
                h1.wait()
                h2.wait()

                def edge_zero(i, _):
                    txbuf[i, pl.ds(0, 16)] = zero16
                    txbuf[i, pl.ds(16, 16)] = zero16
                    tpbuf[i, pl.ds(0, 16)] = zero16
                    return _

                def px4(t, _):
                    for j in range(4):
                        i = t * 4 + j
                        col = jnp.clip(i + s, 0, _XIN - 1)
                        cols = jnp.full((16,), col, jnp.int32)
                        v0 = plsc.load_gather(xbuf, [lanes, cols])
                        v1 = plsc.load_gather(xbuf, [lanes + 16, cols])
                        vp = plsc.load_gather(pbuf, [lanes & 3, cols])
                        txbuf[i, pl.ds(0, 16)] = v0
                        txbuf[i, pl.ds(16, 16)] = v1
                        tpbuf[i, pl.ds(0, 16)] = jnp.where(lanes < 4, vp, 0.0)
                    return _

                lax.fori_loop(0, _XCHUNK // 4, px4, None)
                lax.fori_loop(0, ilo, edge_zero, None)
                lax.fori_loop(ihi, _XCHUNK, edge_zero, None)
                pltpu.sync_copy(txbuf, tx_hbm.at[pl.ds(p0, _XCHUNK)])
                pltpu.sync_copy(tpbuf, tp_hbm.at[pl.ds(p0, _XCHUNK)])
            return _

        lax.fori_loop(0, nunits, unit, None)

    return k(x3, p3)


def _sc_gather(idx2, tx, tp):
    rows = idx2.shape[0] * _IDX_MINOR
    rows_per_tile = rows // _NTILES
    chunks_per_tile = rows_per_tile // _CHUNK
    mesh = plsc.VectorSubcoreMesh(core_axis_name="c", subcore_axis_name="s")

    @functools.partial(
        pl.kernel,
        mesh=mesh,
        compiler_params=pltpu.CompilerParams(use_tc_tiling_on_sc=False),
        out_type=[
            jax.ShapeDtypeStruct((rows, _C), jnp.float32),
            jax.ShapeDtypeStruct((rows, _C), jnp.float32),
        ],
        scratch_types=[
            pltpu.VMEM((_CHUNK_GROUPS, _IDX_MINOR), jnp.int32),
            pltpu.VMEM((_CHUNK, _C), jnp.float32),
            pltpu.VMEM((_CHUNK, _C), jnp.float32),
            pltpu.SemaphoreType.DMA,
        ],
    )
    def k(idx_hbm, tx_hbm, tp_hbm, gx_hbm, gp_hbm, idx_v, gx_v, gp_v, sem):
        wid = lax.axis_index("s") * 2 + lax.axis_index("c")
        idx_row0 = wid * (rows_per_tile // _IDX_MINOR)
        row0 = wid * rows_per_tile

        def body(c, _):
            pltpu.sync_copy(
                idx_hbm.at[pl.ds(idx_row0 + c * _CHUNK_GROUPS, _CHUNK_GROUPS)],
                idx_v)
            handles = []
            for j in range(_CHUNK_GROUPS):
                handles.append(pltpu.async_copy(
                    tx_hbm.at[idx_v.at[j]],
                    gx_v.at[pl.ds(j * _IDX_MINOR, _IDX_MINOR)], sem))
                handles.append(pltpu.async_copy(
                    tp_hbm.at[idx_v.at[j]],
                    gp_v.at[pl.ds(j * _IDX_MINOR, _IDX_MINOR)], sem))
            for h in handles:
                h.wait()
            base = row0 + c * _CHUNK
            pltpu.sync_copy(gx_v, gx_hbm.at[pl.ds(base, _CHUNK)])
            pltpu.sync_copy(gp_v, gp_hbm.at[pl.ds(base, _CHUNK)])
            return _

        lax.fori_loop(0, chunks_per_tile, body, None)

    return k(idx2, tx, tp)


def _tc_body(gx_ref, gp_ref, up_ref, fc1_ref, b1_ref, lnw_ref, lnb_ref,
             g_ref, bg_ref, fc2_ref, b2_ref, s_ref, bs_ref, conv_ref, cb_ref,
             out_ref):
    gp3 = gp_ref[...]
    up = up_ref[...]
    d = jnp.abs(gp3 - up[None, :, :]).reshape(_R, 128)
    hc = jnp.dot(d, fc1_ref[...], preferred_element_type=jnp.float32) + b1_ref[...]
    vs = jnp.dot(hc * hc, g_ref[...], preferred_element_type=jnp.float32)
    inv = jnp.dot(lax.rsqrt(vs + 1e-5), bg_ref[...],
                  preferred_element_type=jnp.float32)
    h = hc * inv * lnw_ref[...] + lnb_ref[...]
    h = 0.5 * h * (1.0 + lax.erf(h * 0.7071067811865476))
    h = jnp.dot(h, fc2_ref[...], preferred_element_type=jnp.float32) + b2_ref[...]
    e = jnp.exp(h)
    s = jnp.dot(e, s_ref[...], preferred_element_type=jnp.float32)
    w = e * jnp.dot(1.0 / s, bs_ref[...], preferred_element_type=jnp.float32)
    wx = gx_ref[...].reshape(_R, 128) * w
    wx3 = wx.reshape(_K, _QBLK, 128)
    res = lax.dot_general(
        wx3, conv_ref[...],
        dimension_numbers=(((2,), (1,)), ((0,), (0,))),
        preferred_element_type=jnp.float32)
    out_ref[...] = jnp.sum(res, axis=0) + cb_ref[...]


def _tc_mlp(gx3, gp3, up128, weights):
    q = up128.shape[0]
    return pl.pallas_call(
        _tc_body,
        grid=(q // _QBLK,),
        in_specs=[
            pl.BlockSpec((_K, _QBLK, 128), lambda i: (0, i, 0)),
            pl.BlockSpec((_K, _QBLK, 128), lambda i: (0, i, 0)),
            pl.BlockSpec((_QBLK, 128), lambda i: (i, 0)),
            pl.BlockSpec((128, 256), lambda i: (0, 0)),
            pl.BlockSpec((256,), lambda i: (0,)),
            pl.BlockSpec((256,), lambda i: (0,)),
            pl.BlockSpec((256,), lambda i: (0,)),
            pl.BlockSpec((256, _PPACK), lambda i: (0, 0)),
            pl.BlockSpec((_PPACK, 256), lambda i: (0, 0)),
            pl.BlockSpec((256, 128), lambda i: (0, 0)),
            pl.BlockSpec((128,), lambda i: (0,)),
            pl.BlockSpec((128, _PPACK), lambda i: (0, 0)),
            pl.BlockSpec((_PPACK, 128), lambda i: (0, 0)),
            pl.BlockSpec((_K, 128, _OL), lambda i: (0, 0, 0)),
            pl.BlockSpec((1, _OL), lambda i: (0, 0)),
        ],
        out_specs=pl.BlockSpec((_QBLK, _OL), lambda i: (i, 0)),
        out_shape=jax.ShapeDtypeStruct((q, _OL), jnp.float32),
    )(gx3, gp3, up128, *weights)


def kernel(x, proj_range_xyz, unproj_range_xyz, p2ri_lut, num_valid_pts,
           fc1_w, fc1_b, ln_w, ln_b, fc2_w, fc2_b, conv_w, conv_b):
    f32 = jnp.float32
    tx, tp = _sc_build_tables(x[0], proj_range_xyz[0])

    lut = p2ri_lut[0]
    yc = lut[:, 1]
    xc = lut[:, 2]
    dy = jnp.arange(_S, dtype=jnp.int32)
    off = (dy[:, None] * _WPP + dy[None, :]).reshape(_K)
    base = yc * _WPP + xc
    idx = off[:, None] + base[None, :]

    up128 = jnp.pad(unproj_range_xyz[0], ((0, 0), (0, _C - 4))).reshape(-1, 128)

    eye4 = jnp.eye(_PPACK, dtype=f32)
    fc1p = jnp.pad(fc1_w.T, ((0, _C - 4), (0, 0)))
    fc1c = fc1p - jnp.mean(fc1p, axis=1, keepdims=True)
    fc1bd = jnp.kron(eye4, fc1c)
    b1c = jnp.tile(fc1_b - jnp.mean(fc1_b), _PPACK)
    lnw = jnp.tile(ln_w, _PPACK)
    lnb = jnp.tile(ln_b, _PPACK)
    gmat = jnp.kron(eye4, jnp.full((64, 1), 1.0 / 64, f32))
    bg4 = jnp.kron(eye4, jnp.ones((1, 64), f32))
    fc2bd = jnp.kron(eye4, fc2_w.T)
    b2t = jnp.tile(fc2_b, _PPACK)
    smat = jnp.kron(eye4, jnp.ones((_C, 1), f32))
    bs4 = jnp.kron(eye4, jnp.ones((1, _C), f32))
    convr3 = jnp.transpose(conv_w.reshape(_NCLASSES, _C, _K), (2, 1, 0))
    conv4 = jax.vmap(lambda m: jnp.kron(eye4, m))(convr3)
    cb = jnp.tile(conv_b, _PPACK)[None]
    weights = (fc1bd, b1c, lnw, lnb, gmat, bg4, fc2bd, b2t, smat, bs4,
               conv4, cb)

    outs = []
    for a, b in ((0, 4096), (4096, 10240), (10240, _N)):
        idx2s = idx[:, a:b].reshape(-1, _IDX_MINOR).astype(jnp.int32)
        gxs, gps = _sc_gather(idx2s, tx, tp)
        qn = (b - a) // _PPACK
        outs.append(_tc_mlp(gxs.reshape(_K, qn, 128),
                            gps.reshape(_K, qn, 128),
                            up128[a // _PPACK:b // _PPACK], weights))
    out = jnp.concatenate(outs, axis=0)
    out = out.reshape(-1, _PPACK, _NCLASSES).transpose(2, 0, 1).reshape(
        _NCLASSES, _N)
    return out[None]
